# Initial kernel scaffold; baseline (speedup 1.0000x reference)
import jax
import jax.numpy as jnp
from jax.experimental import pallas as pl


def kernel(X, edge_index, neg_edge, W, b):
    raise NotImplementedError("write your pallas kernel here")


# speedup vs baseline: 7.2894x; 7.2894x over previous
import functools

import jax
import jax.numpy as jnp
from jax import lax
from jax.experimental import pallas as pl
from jax.experimental.pallas import tpu as pltpu
from jax.experimental.pallas import tpu_sc as plsc

_N = 10000
_E = 320000
_D = 128
_K = 3
_ALPHA = 0.1
_TEMP = 0.5
_TYPE_SIZES = (4000, 3000, 3000)

_NC, _NS = 2, 16
_NW = _NC * _NS
_EB = 80
_EPW = _E // _NW
_NB = _EPW // _EB
_RPS = 632
_NP = _NS * _RPS

_mesh = plsc.VectorSubcoreMesh(
    core_axis_name="c", subcore_axis_name="s",
    num_cores=_NC, num_subcores=_NS)


def _prep_body(x_ref, w_ref, b_ref, h_ref, z_ref, mean_ref, std_ref):
    h = jnp.dot(x_ref[...], w_ref[...], preferred_element_type=jnp.float32)
    h = h + b_ref[...]
    nrm = jnp.sqrt(jnp.sum(h * h, axis=1, keepdims=True))
    h = h / jnp.maximum(nrm, 1e-12)
    means, stds, rows = [], [], []
    start = 0
    for s in _TYPE_SIZES:
        seg = h[start:start + s]
        m = jnp.sum(seg, axis=0, keepdims=True) / s
        v = jnp.sum((seg - m) ** 2, axis=0, keepdims=True) / (s - 1)
        sd = jnp.sqrt(v)
        means.append(m)
        stds.append(sd)
        rows.append((seg - m) / sd)
        start += s
    tilde = jnp.concatenate(rows, axis=0)
    h_ref[...] = tilde
    zn = jnp.sqrt(jnp.sum(tilde * tilde, axis=1, keepdims=True))
    z_ref[...] = tilde / jnp.maximum(zn, 1e-12)
    pad = jnp.zeros((8 - len(_TYPE_SIZES), _D), dtype=jnp.float32)
    mean_ref[...] = jnp.concatenate(means + [pad], axis=0)
    std_ref[...] = jnp.concatenate(stds + [pad], axis=0)


_prep = pl.pallas_call(
    _prep_body,
    out_shape=[
        jax.ShapeDtypeStruct((_N, _D), jnp.float32),
        jax.ShapeDtypeStruct((_N, _D), jnp.float32),
        jax.ShapeDtypeStruct((8, _D), jnp.float32),
        jax.ShapeDtypeStruct((8, _D), jnp.float32),
    ],
)


_DL = 128


@functools.partial(
    pl.kernel,
    out_type=jax.ShapeDtypeStruct((_NC, _NS, _RPS, _DL), jnp.float32),
    mesh=_mesh,
    scratch_types=[
        pltpu.VMEM((_EB,), jnp.int32),
        pltpu.VMEM((_EB, _DL), jnp.float32),
        pltpu.VMEM_SHARED((_NP, _DL), jnp.float32),
        pltpu.SemaphoreType.DMA,
    ],
)
def _deg_sc(col_hbm, ones_hbm, zeros_hbm, out_hbm, col_v, ones_v, deg_sh, sem):
    cid = lax.axis_index("c")
    sid = lax.axis_index("s")
    wid = sid * _NC + cid
    pltpu.sync_copy(zeros_hbm, deg_sh.at[pl.ds(sid * _RPS, _RPS)])
    pltpu.sync_copy(ones_hbm, ones_v)
    plsc.subcore_barrier()

    def body(i, carry):
        base = wid * _EPW + i * _EB
        pltpu.sync_copy(col_hbm.at[pl.ds(base, _EB)], col_v)
        pltpu.sync_copy(ones_v, deg_sh.at[col_v], add=True)
        return carry

    lax.fori_loop(0, _NB, body, 0)
    plsc.subcore_barrier()
    pltpu.sync_copy(deg_sh.at[pl.ds(sid * _RPS, _RPS)], out_hbm.at[cid, sid])


@functools.partial(
    pl.kernel,
    out_type=jax.ShapeDtypeStruct((_NC, _NS, _RPS, _D), jnp.float32),
    mesh=_mesh,
    scratch_types=[
        pltpu.VMEM((_EB,), jnp.int32),
        pltpu.VMEM((_EB,), jnp.int32),
        pltpu.VMEM((_EB, _D), jnp.float32),
        pltpu.VMEM_SHARED((_NP, _D), jnp.float32),
        pltpu.SemaphoreType.DMA,
    ],
)
def _scatter_sc(y_hbm, row_hbm, col_hbm, zeros_hbm, out_hbm,
                row_v, col_v, rows_v, agg_sh, sem):
    cid = lax.axis_index("c")
    sid = lax.axis_index("s")
    wid = sid * _NC + cid
    pltpu.sync_copy(zeros_hbm, agg_sh.at[pl.ds(sid * _RPS, _RPS)])
    plsc.subcore_barrier()

    def body(i, carry):
        base = wid * _EPW + i * _EB
        pltpu.sync_copy(row_hbm.at[pl.ds(base, _EB)], row_v)
        pltpu.sync_copy(col_hbm.at[pl.ds(base, _EB)], col_v)
        pltpu.async_copy(y_hbm.at[row_v], rows_v, sem).wait()
        pltpu.sync_copy(rows_v, agg_sh.at[col_v], add=True)
        return carry

    lax.fori_loop(0, _NB, body, 0)
    plsc.subcore_barrier()
    pltpu.sync_copy(agg_sh.at[pl.ds(sid * _RPS, _RPS)], out_hbm.at[cid, sid])


@functools.partial(
    pl.kernel,
    out_type=jax.ShapeDtypeStruct((_E, 16), jnp.float32),
    mesh=_mesh,
    scratch_types=[
        pltpu.VMEM((_EB,), jnp.int32),
        pltpu.VMEM((_EB,), jnp.int32),
        pltpu.VMEM((_EB, _D), jnp.float32),
        pltpu.VMEM((_EB, _D), jnp.float32),
        pltpu.VMEM((_EB, 16), jnp.float32),
        pltpu.SemaphoreType.DMA,
        pltpu.SemaphoreType.DMA,
    ],
)
def _dots_sc(z_hbm, a_hbm, b_hbm, out_hbm,
             ai_v, bi_v, za_v, zb_v, d_v, sem_a, sem_b):
    cid = lax.axis_index("c")
    sid = lax.axis_index("s")
    wid = sid * _NC + cid

    def body(i, carry):
        base = wid * _EPW + i * _EB
        pltpu.sync_copy(a_hbm.at[pl.ds(base, _EB)], ai_v)
        pltpu.sync_copy(b_hbm.at[pl.ds(base, _EB)], bi_v)
        ca = pltpu.async_copy(z_hbm.at[ai_v], za_v, sem_a)
        cb = pltpu.async_copy(z_hbm.at[bi_v], zb_v, sem_b)
        ca.wait()
        cb.wait()

        def ebody(e, c2):
            acc = za_v[e, pl.ds(0, 16)] * zb_v[e, pl.ds(0, 16)]
            for j in range(1, _D // 16):
                acc = acc + za_v[e, pl.ds(16 * j, 16)] * zb_v[e, pl.ds(16 * j, 16)]
            d_v[e, :] = acc
            return c2

        lax.fori_loop(0, _EB, ebody, 0)
        pltpu.sync_copy(d_v, out_hbm.at[pl.ds(base, _EB)])
        return carry

    lax.fori_loop(0, _NB, body, 0)


def _dinvy_body(degp_ref, h_ref, dinv_ref, y_ref):
    d0 = degp_ref[0].reshape(_NP, _DL)[: _N, 0:1]
    d1 = degp_ref[1].reshape(_NP, _DL)[: _N, 0:1]
    deg = d0 + d1 + 1.0
    dinv = 1.0 / jnp.sqrt(jnp.maximum(deg, 1e-12))
    dinv_ref[...] = jnp.broadcast_to(dinv, (_N, 8))
    y_ref[...] = h_ref[...] * dinv


_dinvy = pl.pallas_call(
    _dinvy_body,
    out_shape=[
        jax.ShapeDtypeStruct((_N, 8), jnp.float32),
        jax.ShapeDtypeStruct((_N, _D), jnp.float32),
    ],
)


def _combine_mid_body(u_ref, y_ref, h_ref, dinv_ref, yn_ref):
    dinv = dinv_ref[:, 0:1]
    u = (u_ref[0].reshape(_NP, _D)[: _N] + u_ref[1].reshape(_NP, _D)[: _N])
    xn = (1.0 - _ALPHA) * (dinv * (u + y_ref[...])) + _ALPHA * h_ref[...]
    yn_ref[...] = xn * dinv


_combine_mid = pl.pallas_call(
    _combine_mid_body,
    out_shape=jax.ShapeDtypeStruct((_N, _D), jnp.float32),
)


def _combine_last_body(u_ref, y_ref, h_ref, dinv_ref, mean_ref, std_ref, z_ref):
    dinv = dinv_ref[:, 0:1]
    u = (u_ref[0].reshape(_NP, _D)[: _N] + u_ref[1].reshape(_NP, _D)[: _N])
    xn = (1.0 - _ALPHA) * (dinv * (u + y_ref[...])) + _ALPHA * h_ref[...]
    parts = []
    start = 0
    for i, s in enumerate(_TYPE_SIZES):
        seg = xn[start:start + s]
        parts.append(seg * std_ref[i:i + 1, :] + mean_ref[i:i + 1, :])
        start += s
    z_ref[...] = jnp.concatenate(parts, axis=0)


_combine_last = pl.pallas_call(
    _combine_last_body,
    out_shape=jax.ShapeDtypeStruct((_N, _D), jnp.float32),
)


_LB = 8000


def _loss_body(dp_ref, dn_ref, loss_ref, acc_ref):
    i = pl.program_id(0)

    @pl.when(i == 0)
    def _():
        acc_ref[0] = 0.0
        acc_ref[1] = 0.0

    acc_ref[0] += jnp.sum(jnp.exp(jnp.sum(dp_ref[...], axis=1) / _TEMP))
    acc_ref[1] += jnp.sum(jnp.exp(jnp.sum(dn_ref[...], axis=1) / _TEMP))

    @pl.when(i == pl.num_programs(0) - 1)
    def _():
        pos = acc_ref[0]
        neg = acc_ref[1]
        loss_ref[...] = jnp.full((1, 1), -jnp.log(pos / (pos + neg)), jnp.float32)


_loss = pl.pallas_call(
    _loss_body,
    grid=(_E // _LB,),
    in_specs=[
        pl.BlockSpec((_LB, 16), lambda i: (i, 0)),
        pl.BlockSpec((_LB, 16), lambda i: (i, 0)),
    ],
    out_specs=pl.BlockSpec((1, 1), lambda i: (0, 0)),
    out_shape=jax.ShapeDtypeStruct((1, 1), jnp.float32),
    scratch_shapes=[pltpu.SMEM((2,), jnp.float32)],
)


@jax.jit
def kernel(X, edge_index, neg_edge, W, b):
    row = edge_index[0]
    col = edge_index[1]
    zeros8 = jnp.zeros((_RPS, _DL), jnp.float32)
    zerosd = jnp.zeros((_RPS, _D), jnp.float32)
    ones8 = jnp.ones((_EB, _DL), jnp.float32)

    h, z, mean8, std8 = _prep(X, W, b.reshape(1, _D))
    degp = _deg_sc(col, ones8, zeros8)
    dinv8, y = _dinvy(degp, h)

    for step in range(_K):
        u = _scatter_sc(y, row, col, zerosd)
        if step < _K - 1:
            y = _combine_mid(u, y, h, dinv8)
        else:
            Z = _combine_last(u, y, h, dinv8, mean8, std8)

    dp = _dots_sc(z, row, col)
    dn = _dots_sc(z, neg_edge[0], neg_edge[1])
    loss = _loss(dp, dn)[0, 0]
    return (Z, loss)


# speedup vs baseline: 7.9543x; 1.0912x over previous
import functools

import jax
import jax.numpy as jnp
from jax import lax
from jax.experimental import pallas as pl
from jax.experimental.pallas import tpu as pltpu
from jax.experimental.pallas import tpu_sc as plsc

_N = 10000
_E = 320000
_D = 128
_K = 3
_ALPHA = 0.1
_TEMP = 0.5
_TYPE_SIZES = (4000, 3000, 3000)

_NC, _NS = 2, 16
_NW = _NC * _NS
_EB = 40
_EPW = _E // _NW
_NB = _EPW // _EB
_EBD = 80
_NBD = _EPW // _EBD
_RPS = 632
_NP = _NS * _RPS

_mesh = plsc.VectorSubcoreMesh(
    core_axis_name="c", subcore_axis_name="s",
    num_cores=_NC, num_subcores=_NS)


def _prep_body(x_ref, w_ref, b_ref, h_ref, z_ref, mean_ref, std_ref):
    h = jnp.dot(x_ref[...], w_ref[...], preferred_element_type=jnp.float32)
    h = h + b_ref[...]
    nrm = jnp.sqrt(jnp.sum(h * h, axis=1, keepdims=True))
    h = h / jnp.maximum(nrm, 1e-12)
    means, stds, rows = [], [], []
    start = 0
    for s in _TYPE_SIZES:
        seg = h[start:start + s]
        m = jnp.sum(seg, axis=0, keepdims=True) / s
        v = jnp.sum((seg - m) ** 2, axis=0, keepdims=True) / (s - 1)
        sd = jnp.sqrt(v)
        means.append(m)
        stds.append(sd)
        rows.append((seg - m) / sd)
        start += s
    tilde = jnp.concatenate(rows, axis=0)
    h_ref[...] = tilde
    zn = jnp.sqrt(jnp.sum(tilde * tilde, axis=1, keepdims=True))
    z_ref[...] = tilde / jnp.maximum(zn, 1e-12)
    pad = jnp.zeros((8 - len(_TYPE_SIZES), _D), dtype=jnp.float32)
    mean_ref[...] = jnp.concatenate(means + [pad], axis=0)
    std_ref[...] = jnp.concatenate(stds + [pad], axis=0)


_prep = pl.pallas_call(
    _prep_body,
    out_shape=[
        jax.ShapeDtypeStruct((_N, _D), jnp.float32),
        jax.ShapeDtypeStruct((_N, _D), jnp.float32),
        jax.ShapeDtypeStruct((8, _D), jnp.float32),
        jax.ShapeDtypeStruct((8, _D), jnp.float32),
    ],
)


_DL = 128


@functools.partial(
    pl.kernel,
    out_type=jax.ShapeDtypeStruct((_NC, _NS, _RPS, _DL), jnp.float32),
    mesh=_mesh,
    scratch_types=[
        pltpu.VMEM((_EBD,), jnp.int32),
        pltpu.VMEM((_EBD, _DL), jnp.float32),
        pltpu.VMEM_SHARED((_NP, _DL), jnp.float32),
        pltpu.SemaphoreType.DMA,
    ],
)
def _deg_sc(col_hbm, ones_hbm, zeros_hbm, out_hbm, col_v, ones_v, deg_sh, sem):
    cid = lax.axis_index("c")
    sid = lax.axis_index("s")
    wid = sid * _NC + cid
    pltpu.sync_copy(zeros_hbm, deg_sh.at[pl.ds(sid * _RPS, _RPS)])
    pltpu.sync_copy(ones_hbm, ones_v)
    plsc.subcore_barrier()

    def body(i, carry):
        base = wid * _EPW + i * _EBD
        pltpu.sync_copy(col_hbm.at[pl.ds(base, _EBD)], col_v)
        pltpu.sync_copy(ones_v, deg_sh.at[col_v], add=True)
        return carry

    lax.fori_loop(0, _NBD, body, 0)
    plsc.subcore_barrier()
    pltpu.sync_copy(deg_sh.at[pl.ds(sid * _RPS, _RPS)], out_hbm.at[cid, sid])


@functools.partial(
    pl.kernel,
    out_type=jax.ShapeDtypeStruct((_NC, _NS, _RPS, _D), jnp.float32),
    mesh=_mesh,
    scratch_types=[
        pltpu.VMEM((_EB,), jnp.int32),
        pltpu.VMEM((_EB,), jnp.int32),
        pltpu.VMEM((_EB,), jnp.int32),
        pltpu.VMEM((_EB,), jnp.int32),
        pltpu.VMEM((_EB, _D), jnp.float32),
        pltpu.VMEM((_EB, _D), jnp.float32),
        pltpu.VMEM_SHARED((_NP, _D), jnp.float32),
        pltpu.SemaphoreType.DMA,
        pltpu.SemaphoreType.DMA,
    ],
)
def _scatter_sc(y_hbm, row_hbm, col_hbm, zeros_hbm, out_hbm,
                row0_v, col0_v, row1_v, col1_v, rows0_v, rows1_v,
                agg_sh, sem0, sem1):
    cid = lax.axis_index("c")
    sid = lax.axis_index("s")
    wid = sid * _NC + cid
    pltpu.sync_copy(zeros_hbm, agg_sh.at[pl.ds(sid * _RPS, _RPS)])
    plsc.subcore_barrier()
    ebase = wid * _EPW

    def fetch(blk, row_v, col_v, rows_v, sem):
        base = ebase + blk * _EB
        pltpu.sync_copy(row_hbm.at[pl.ds(base, _EB)], row_v)
        pltpu.sync_copy(col_hbm.at[pl.ds(base, _EB)], col_v)
        pltpu.async_copy(y_hbm.at[row_v], rows_v, sem)

    def drain_add(row_v, col_v, rows_v, sem):
        pltpu.make_async_copy(y_hbm.at[row_v], rows_v, sem).wait()
        pltpu.sync_copy(rows_v, agg_sh.at[col_v], add=True)

    fetch(0, row0_v, col0_v, rows0_v, sem0)

    def body(j, carry):
        fetch(2 * j + 1, row1_v, col1_v, rows1_v, sem1)
        drain_add(row0_v, col0_v, rows0_v, sem0)
        fetch(2 * j + 2, row0_v, col0_v, rows0_v, sem0)
        drain_add(row1_v, col1_v, rows1_v, sem1)
        return carry

    lax.fori_loop(0, _NB // 2 - 1, body, 0)
    fetch(_NB - 1, row1_v, col1_v, rows1_v, sem1)
    drain_add(row0_v, col0_v, rows0_v, sem0)
    drain_add(row1_v, col1_v, rows1_v, sem1)
    plsc.subcore_barrier()
    pltpu.sync_copy(agg_sh.at[pl.ds(sid * _RPS, _RPS)], out_hbm.at[cid, sid])


@functools.partial(
    pl.kernel,
    out_type=jax.ShapeDtypeStruct((_E, 16), jnp.float32),
    mesh=_mesh,
    scratch_types=[
        pltpu.VMEM((_EB,), jnp.int32),
        pltpu.VMEM((_EB,), jnp.int32),
        pltpu.VMEM((_EB,), jnp.int32),
        pltpu.VMEM((_EB,), jnp.int32),
        pltpu.VMEM((_EB, _D), jnp.float32),
        pltpu.VMEM((_EB, _D), jnp.float32),
        pltpu.VMEM((_EB, _D), jnp.float32),
        pltpu.VMEM((_EB, _D), jnp.float32),
        pltpu.VMEM((_EB, 16), jnp.float32),
        pltpu.SemaphoreType.DMA,
        pltpu.SemaphoreType.DMA,
    ],
)
def _dots_sc(z_hbm, a_hbm, b_hbm, out_hbm,
             ai0_v, bi0_v, ai1_v, bi1_v, za0_v, zb0_v, za1_v, zb1_v,
             d_v, sem0, sem1):
    cid = lax.axis_index("c")
    sid = lax.axis_index("s")
    wid = sid * _NC + cid
    ebase = wid * _EPW

    def fetch(blk, ai_v, bi_v, za_v, zb_v, sem):
        base = ebase + blk * _EB
        pltpu.sync_copy(a_hbm.at[pl.ds(base, _EB)], ai_v)
        pltpu.sync_copy(b_hbm.at[pl.ds(base, _EB)], bi_v)
        pltpu.async_copy(z_hbm.at[ai_v], za_v, sem)
        pltpu.async_copy(z_hbm.at[bi_v], zb_v, sem)

    def drain_dot(blk, ai_v, bi_v, za_v, zb_v, sem):
        pltpu.make_async_copy(z_hbm.at[ai_v], za_v, sem).wait()
        pltpu.make_async_copy(z_hbm.at[bi_v], zb_v, sem).wait()

        def ebody(e, c2):
            acc = za_v[e, pl.ds(0, 16)] * zb_v[e, pl.ds(0, 16)]
            for j in range(1, _D // 16):
                acc = acc + za_v[e, pl.ds(16 * j, 16)] * zb_v[e, pl.ds(16 * j, 16)]
            d_v[e, :] = acc
            return c2

        lax.fori_loop(0, _EB, ebody, 0)
        pltpu.sync_copy(d_v, out_hbm.at[pl.ds(ebase + blk * _EB, _EB)])

    fetch(0, ai0_v, bi0_v, za0_v, zb0_v, sem0)

    def body(j, carry):
        fetch(2 * j + 1, ai1_v, bi1_v, za1_v, zb1_v, sem1)
        drain_dot(2 * j, ai0_v, bi0_v, za0_v, zb0_v, sem0)
        fetch(2 * j + 2, ai0_v, bi0_v, za0_v, zb0_v, sem0)
        drain_dot(2 * j + 1, ai1_v, bi1_v, za1_v, zb1_v, sem1)
        return carry

    lax.fori_loop(0, _NB // 2 - 1, body, 0)
    fetch(_NB - 1, ai1_v, bi1_v, za1_v, zb1_v, sem1)
    drain_dot(_NB - 2, ai0_v, bi0_v, za0_v, zb0_v, sem0)
    drain_dot(_NB - 1, ai1_v, bi1_v, za1_v, zb1_v, sem1)


def _dinvy_body(degp_ref, h_ref, dinv_ref, y_ref):
    d0 = degp_ref[0].reshape(_NP, _DL)[: _N, 0:1]
    d1 = degp_ref[1].reshape(_NP, _DL)[: _N, 0:1]
    deg = d0 + d1 + 1.0
    dinv = 1.0 / jnp.sqrt(jnp.maximum(deg, 1e-12))
    dinv_ref[...] = jnp.broadcast_to(dinv, (_N, 8))
    y_ref[...] = h_ref[...] * dinv


_dinvy = pl.pallas_call(
    _dinvy_body,
    out_shape=[
        jax.ShapeDtypeStruct((_N, 8), jnp.float32),
        jax.ShapeDtypeStruct((_N, _D), jnp.float32),
    ],
)


def _combine_mid_body(u_ref, y_ref, h_ref, dinv_ref, yn_ref):
    dinv = dinv_ref[:, 0:1]
    u = (u_ref[0].reshape(_NP, _D)[: _N] + u_ref[1].reshape(_NP, _D)[: _N])
    xn = (1.0 - _ALPHA) * (dinv * (u + y_ref[...])) + _ALPHA * h_ref[...]
    yn_ref[...] = xn * dinv


_combine_mid = pl.pallas_call(
    _combine_mid_body,
    out_shape=jax.ShapeDtypeStruct((_N, _D), jnp.float32),
)


def _combine_last_body(u_ref, y_ref, h_ref, dinv_ref, mean_ref, std_ref, z_ref):
    dinv = dinv_ref[:, 0:1]
    u = (u_ref[0].reshape(_NP, _D)[: _N] + u_ref[1].reshape(_NP, _D)[: _N])
    xn = (1.0 - _ALPHA) * (dinv * (u + y_ref[...])) + _ALPHA * h_ref[...]
    parts = []
    start = 0
    for i, s in enumerate(_TYPE_SIZES):
        seg = xn[start:start + s]
        parts.append(seg * std_ref[i:i + 1, :] + mean_ref[i:i + 1, :])
        start += s
    z_ref[...] = jnp.concatenate(parts, axis=0)


_combine_last = pl.pallas_call(
    _combine_last_body,
    out_shape=jax.ShapeDtypeStruct((_N, _D), jnp.float32),
)


_LB = 8000


def _loss_body(dp_ref, dn_ref, loss_ref, acc_ref):
    i = pl.program_id(0)

    @pl.when(i == 0)
    def _():
        acc_ref[0] = 0.0
        acc_ref[1] = 0.0

    acc_ref[0] += jnp.sum(jnp.exp(jnp.sum(dp_ref[...], axis=1) / _TEMP))
    acc_ref[1] += jnp.sum(jnp.exp(jnp.sum(dn_ref[...], axis=1) / _TEMP))

    @pl.when(i == pl.num_programs(0) - 1)
    def _():
        pos = acc_ref[0]
        neg = acc_ref[1]
        loss_ref[...] = jnp.full((1, 1), -jnp.log(pos / (pos + neg)), jnp.float32)


_loss = pl.pallas_call(
    _loss_body,
    grid=(_E // _LB,),
    in_specs=[
        pl.BlockSpec((_LB, 16), lambda i: (i, 0)),
        pl.BlockSpec((_LB, 16), lambda i: (i, 0)),
    ],
    out_specs=pl.BlockSpec((1, 1), lambda i: (0, 0)),
    out_shape=jax.ShapeDtypeStruct((1, 1), jnp.float32),
    scratch_shapes=[pltpu.SMEM((2,), jnp.float32)],
)


@jax.jit
def kernel(X, edge_index, neg_edge, W, b):
    row = edge_index[0]
    col = edge_index[1]
    zeros8 = jnp.zeros((_RPS, _DL), jnp.float32)
    zerosd = jnp.zeros((_RPS, _D), jnp.float32)
    ones8 = jnp.ones((_EBD, _DL), jnp.float32)

    h, z, mean8, std8 = _prep(X, W, b.reshape(1, _D))
    degp = _deg_sc(col, ones8, zeros8)
    dinv8, y = _dinvy(degp, h)

    for step in range(_K):
        u = _scatter_sc(y, row, col, zerosd)
        if step < _K - 1:
            y = _combine_mid(u, y, h, dinv8)
        else:
            Z = _combine_last(u, y, h, dinv8, mean8, std8)

    dp = _dots_sc(z, row, col)
    dn = _dots_sc(z, neg_edge[0], neg_edge[1])
    loss = _loss(dp, dn)[0, 0]
    return (Z, loss)
